# baseline (speedup 1.0000x reference)
import functools

import jax
import jax.numpy as jnp
from jax import lax
from jax.experimental import pallas as pl
from jax.experimental.pallas import tpu as pltpu

N = 10000
E = 320000
D_IN = 128
D_OUT = 256
D_EDGE = 16

_S3 = 3.0 ** 0.5
_S5 = 5.0 ** 0.5
_S15 = 15.0 ** 0.5


def _node_mlp_body(x_ref, w1_ref, b1_ref, w2_ref, b2_ref, out_ref):
    h = jax.nn.silu(
        jnp.dot(x_ref[...], w1_ref[...], preferred_element_type=jnp.float32)
        + b1_ref[...]
    )
    out_ref[...] = (
        jnp.dot(h, w2_ref[...], preferred_element_type=jnp.float32) + b2_ref[...]
    )


def _node_mlp(x, Wn1, bn1, Wn2, bn2):
    return pl.pallas_call(
        _node_mlp_body,
        out_shape=jax.ShapeDtypeStruct((N, D_OUT), jnp.float32),
    )(x, Wn1, bn1.reshape(1, -1), Wn2, bn2.reshape(1, -1))


def _edge_body(rel_ref, ea_ref, xtg_ref, w1a_ref, w1b_ref, b1_ref, w2_ref,
               b2_ref, out_ref):
    rel = rel_ref[...]
    rx = rel[:, 0:1]
    ry = rel[:, 1:2]
    rz = rel[:, 2:3]
    r2 = rx * rx + ry * ry + rz * rz
    inv = 1.0 / (jnp.sqrt(r2) + 1e-8)
    xd = rx * inv
    yd = ry * inv
    zd = rz * inv
    w = w1a_ref[...]
    h = b1_ref[...] + w[0:1, :]
    h = h + (_S3 * xd) * w[1:2, :]
    h = h + (_S3 * yd) * w[2:3, :]
    h = h + (_S3 * zd) * w[3:4, :]
    h = h + (_S15 * xd * yd) * w[4:5, :]
    h = h + (_S15 * yd * zd) * w[5:6, :]
    h = h + ((_S5 / 2.0) * (3.0 * zd * zd - 1.0)) * w[6:7, :]
    h = h + (_S15 * xd * zd) * w[7:8, :]
    h = h + ((_S15 / 2.0) * (xd * xd - yd * yd)) * w[8:9, :]
    h = h + jnp.dot(ea_ref[...], w1b_ref[...], preferred_element_type=jnp.float32)
    h = jax.nn.silu(h)
    em = jnp.dot(h, w2_ref[...], preferred_element_type=jnp.float32) + b2_ref[...]
    out_ref[...] = xtg_ref[...] * em


def _edge_msg(rel, edge_attr, xt_g, We1, be1, We2, be2):
    B = 2000
    grid = E // B
    return pl.pallas_call(
        _edge_body,
        grid=(grid,),
        in_specs=[
            pl.BlockSpec((B, 3), lambda i: (i, 0)),
            pl.BlockSpec((B, D_EDGE), lambda i: (i, 0)),
            pl.BlockSpec((B, D_OUT), lambda i: (i, 0)),
            pl.BlockSpec((16, D_OUT), lambda i: (0, 0)),
            pl.BlockSpec((D_EDGE, D_OUT), lambda i: (0, 0)),
            pl.BlockSpec((1, D_OUT), lambda i: (0, 0)),
            pl.BlockSpec((D_OUT, D_OUT), lambda i: (0, 0)),
            pl.BlockSpec((1, D_OUT), lambda i: (0, 0)),
        ],
        out_specs=pl.BlockSpec((B, D_OUT), lambda i: (i, 0)),
        out_shape=jax.ShapeDtypeStruct((E, D_OUT), jnp.float32),
    )(rel, edge_attr, xt_g,
      jnp.pad(We1[:9], ((0, 7), (0, 0))), We1[9:], be1.reshape(1, -1),
      We2, be2.reshape(1, -1))


def _out_body(agg_ref, cnt_ref, x_ref, w1a_ref, w1b_ref, b1_ref, w2_ref,
              b2_ref, g_ref, bt_ref, out_ref):
    agg = agg_ref[...] / jnp.maximum(cnt_ref[...], 1.0)
    h = jax.nn.silu(
        jnp.dot(agg, w1a_ref[...], preferred_element_type=jnp.float32)
        + jnp.dot(x_ref[...], w1b_ref[...], preferred_element_type=jnp.float32)
        + b1_ref[...]
    )
    h = jnp.dot(h, w2_ref[...], preferred_element_type=jnp.float32) + b2_ref[...]
    mu = jnp.mean(h, axis=0, keepdims=True)
    var = jnp.mean((h - mu) ** 2, axis=0, keepdims=True)
    out_ref[...] = (h - mu) * lax.rsqrt(var + 1e-5) * g_ref[...] + bt_ref[...]


def _out_mlp(agg, cnt, x, Wo1, bo1, Wo2, bo2, gamma, beta):
    return pl.pallas_call(
        _out_body,
        out_shape=jax.ShapeDtypeStruct((N, D_OUT), jnp.float32),
    )(agg, cnt.reshape(-1, 1), x, Wo1[:D_OUT], Wo1[D_OUT:],
      bo1.reshape(1, -1), Wo2, bo2.reshape(1, -1), gamma.reshape(1, -1),
      beta.reshape(1, -1))


def kernel(x, edge_index, edge_attr, pos, Wn1, bn1, Wn2, bn2, We1, be1, We2,
           be2, Wo1, bo1, Wo2, bo2, gamma, beta):
    row = edge_index[0]
    col = edge_index[1]
    xt = _node_mlp(x, Wn1, bn1, Wn2, bn2)
    rel = jnp.take(pos, row, axis=0) - jnp.take(pos, col, axis=0)
    xt_g = jnp.take(xt, row, axis=0)
    msg = _edge_msg(rel, edge_attr, xt_g, We1, be1, We2, be2)
    agg = jax.ops.segment_sum(msg, col, num_segments=N)
    cnt = jax.ops.segment_sum(jnp.ones((E,), jnp.float32), col, num_segments=N)
    return _out_mlp(agg, cnt, x, Wo1, bo1, Wo2, bo2, gamma, beta)


# speedup vs baseline: 1.5416x; 1.5416x over previous
import functools

import jax
import jax.numpy as jnp
from jax import lax
from jax.experimental import pallas as pl
from jax.experimental.pallas import tpu as pltpu
from jax.experimental.pallas import tpu_sc as plsc

N = 10000
E = 320000
D_IN = 128
D_OUT = 256
DH = 128
D_EDGE = 16

NS = 16
EPT = E // NS
C = 80
NCHUNK = EPT // C
NPT = 640
NPT_STRIDE = 624

_S3 = 3.0 ** 0.5
_S5 = 5.0 ** 0.5
_S15 = 15.0 ** 0.5


def _node_mlp_body(x_ref, w1_ref, b1_ref, w2_ref, b2_ref, lo_ref, hi_ref):
    h = jax.nn.silu(
        jnp.dot(x_ref[...], w1_ref[...], preferred_element_type=jnp.float32)
        + b1_ref[...]
    )
    o = jnp.dot(h, w2_ref[...], preferred_element_type=jnp.float32) + b2_ref[...]
    lo_ref[...] = o[:, :DH]
    hi_ref[...] = o[:, DH:]


def _node_mlp(x, Wn1, bn1, Wn2, bn2):
    return pl.pallas_call(
        _node_mlp_body,
        out_shape=(
            jax.ShapeDtypeStruct((N, DH), jnp.float32),
            jax.ShapeDtypeStruct((N, DH), jnp.float32),
        ),
    )(x, Wn1, bn1.reshape(1, -1), Wn2, bn2.reshape(1, -1))


def _edge_body(rel_ref, ea_ref, w1a_ref, w1b_ref, b1_ref, w2_ref, b2_ref,
               lo_ref, hi_ref):
    rel = rel_ref[...]
    rx = rel[:, 0:1]
    ry = rel[:, 1:2]
    rz = rel[:, 2:3]
    r2 = rx * rx + ry * ry + rz * rz
    inv = 1.0 / (jnp.sqrt(r2) + 1e-8)
    xd = rx * inv
    yd = ry * inv
    zd = rz * inv
    w = w1a_ref[...]
    h = b1_ref[...] + w[0:1, :]
    h = h + (_S3 * xd) * w[1:2, :]
    h = h + (_S3 * yd) * w[2:3, :]
    h = h + (_S3 * zd) * w[3:4, :]
    h = h + (_S15 * xd * yd) * w[4:5, :]
    h = h + (_S15 * yd * zd) * w[5:6, :]
    h = h + ((_S5 / 2.0) * (3.0 * zd * zd - 1.0)) * w[6:7, :]
    h = h + (_S15 * xd * zd) * w[7:8, :]
    h = h + ((_S15 / 2.0) * (xd * xd - yd * yd)) * w[8:9, :]
    h = h + jnp.dot(ea_ref[...], w1b_ref[...], preferred_element_type=jnp.float32)
    h = jax.nn.silu(h)
    em = jnp.dot(h, w2_ref[...], preferred_element_type=jnp.float32) + b2_ref[...]
    lo_ref[...] = em[:, :DH]
    hi_ref[...] = em[:, DH:]


def _edge_msg(rel, edge_attr, We1, be1, We2, be2):
    B = 2000
    grid = E // B
    return pl.pallas_call(
        _edge_body,
        grid=(grid,),
        in_specs=[
            pl.BlockSpec((B, 3), lambda i: (i, 0)),
            pl.BlockSpec((B, D_EDGE), lambda i: (i, 0)),
            pl.BlockSpec((16, D_OUT), lambda i: (0, 0)),
            pl.BlockSpec((D_EDGE, D_OUT), lambda i: (0, 0)),
            pl.BlockSpec((1, D_OUT), lambda i: (0, 0)),
            pl.BlockSpec((D_OUT, D_OUT), lambda i: (0, 0)),
            pl.BlockSpec((1, D_OUT), lambda i: (0, 0)),
        ],
        out_specs=(
            pl.BlockSpec((B, DH), lambda i: (i, 0)),
            pl.BlockSpec((B, DH), lambda i: (i, 0)),
        ),
        out_shape=(
            jax.ShapeDtypeStruct((E, DH), jnp.float32),
            jax.ShapeDtypeStruct((E, DH), jnp.float32),
        ),
    )(rel, edge_attr,
      jnp.pad(We1[:9], ((0, 7), (0, 0))), We1[9:], be1.reshape(1, -1),
      We2, be2.reshape(1, -1))


def _sc_body(emlo, emhi, xtlo, xthi, rowi, coli,
             agglo, agghi, cnto,
             em_v, xt_v, ridx_v, cidx_v, ones_v, zc_v,
             agg_sh, cnt_sh, gsem):
    cid = lax.axis_index("c")
    sid = lax.axis_index("s")
    r0 = sid * NPT_STRIDE

    zeros16 = jnp.zeros((16,), jnp.float32)
    ones16 = jnp.ones((16,), jnp.float32)

    def zrow(i, _):
        for j in range(DH // 16):
            em_v[i, pl.ds(j * 16, 16)] = zeros16
        ones_v[i, :] = ones16
        zc_v[i, :] = zeros16
        return 0

    lax.fori_loop(0, C, zrow, 0)

    for j in range(NPT // C):
        pltpu.sync_copy(em_v, agg_sh.at[pl.ds(r0 + j * C, C)])
        pltpu.sync_copy(zc_v, cnt_sh.at[pl.ds(r0 + j * C, C)])
    plsc.subcore_barrier()

    def run(em_ref, xt_ref, agg_out, do_cnt):
        base = sid * EPT

        def chunk(k, _):
            e0 = base + k * C
            pltpu.sync_copy(rowi.at[pl.ds(e0, C)], ridx_v)
            pltpu.sync_copy(coli.at[pl.ds(e0, C)], cidx_v)
            g = pltpu.async_copy(xt_ref.at[ridx_v], xt_v, gsem)
            pltpu.sync_copy(em_ref.at[pl.ds(e0, C)], em_v)
            g.wait()

            def mrow(i, _):
                for j in range(DH // 16):
                    s = (i, pl.ds(j * 16, 16))
                    em_v[s] = em_v[s] * xt_v[s]
                return 0

            lax.fori_loop(0, C, mrow, 0)
            pltpu.sync_copy(em_v, agg_sh.at[cidx_v], add=True)
            if do_cnt:
                pltpu.sync_copy(ones_v, cnt_sh.at[cidx_v], add=True)
            return 0

        lax.fori_loop(0, NCHUNK, chunk, 0)
        plsc.subcore_barrier()
        for j in range(NPT // C):
            pltpu.sync_copy(agg_sh.at[pl.ds(r0 + j * C, C)], em_v)
            pltpu.sync_copy(em_v, agg_out.at[pl.ds(r0 + j * C, C)])
            if do_cnt:
                pltpu.sync_copy(cnt_sh.at[pl.ds(r0 + j * C, C)], zc_v)
                pltpu.sync_copy(zc_v, cnto.at[pl.ds(r0 + j * C, C)])

    @pl.when(cid == 0)
    def _():
        run(emlo, xtlo, agglo, True)

    @pl.when(cid == 1)
    def _():
        run(emhi, xthi, agghi, False)


def _sc_aggregate(em_lo, em_hi, xt_lo, xt_hi, row, col):
    f = pl.kernel(
        _sc_body,
        out_type=(
            jax.ShapeDtypeStruct((N, DH), jnp.float32),
            jax.ShapeDtypeStruct((N, DH), jnp.float32),
            jax.ShapeDtypeStruct((N, 16), jnp.float32),
        ),
        mesh=plsc.VectorSubcoreMesh(core_axis_name="c", subcore_axis_name="s"),
        compiler_params=pltpu.CompilerParams(use_tc_tiling_on_sc=False),
        scratch_types=[
            pltpu.VMEM((C, DH), jnp.float32),
            pltpu.VMEM((C, DH), jnp.float32),
            pltpu.VMEM((C,), jnp.int32),
            pltpu.VMEM((C,), jnp.int32),
            pltpu.VMEM((C, 16), jnp.float32),
            pltpu.VMEM((C, 16), jnp.float32),
            pltpu.VMEM_SHARED((N, DH), jnp.float32),
            pltpu.VMEM_SHARED((N, 16), jnp.float32),
            pltpu.SemaphoreType.DMA,
        ],
    )
    return f(em_lo, em_hi, xt_lo, xt_hi, row, col)


def _out_body(agglo_ref, agghi_ref, cnt_ref, x_ref, w1a_ref, w1b_ref,
              w1c_ref, b1_ref, w2_ref, b2_ref, g_ref, bt_ref, out_ref):
    inv_cnt = 1.0 / jnp.maximum(cnt_ref[...][:, 0:1], 1.0)
    alo = agglo_ref[...] * inv_cnt
    ahi = agghi_ref[...] * inv_cnt
    h = jax.nn.silu(
        jnp.dot(alo, w1a_ref[...], preferred_element_type=jnp.float32)
        + jnp.dot(ahi, w1b_ref[...], preferred_element_type=jnp.float32)
        + jnp.dot(x_ref[...], w1c_ref[...], preferred_element_type=jnp.float32)
        + b1_ref[...]
    )
    h = jnp.dot(h, w2_ref[...], preferred_element_type=jnp.float32) + b2_ref[...]
    mu = jnp.mean(h, axis=0, keepdims=True)
    var = jnp.mean((h - mu) ** 2, axis=0, keepdims=True)
    out_ref[...] = (h - mu) * lax.rsqrt(var + 1e-5) * g_ref[...] + bt_ref[...]


def _out_mlp(agg_lo, agg_hi, cnt, x, Wo1, bo1, Wo2, bo2, gamma, beta):
    return pl.pallas_call(
        _out_body,
        out_shape=jax.ShapeDtypeStruct((N, D_OUT), jnp.float32),
    )(agg_lo, agg_hi, cnt, x, Wo1[:DH], Wo1[DH:D_OUT], Wo1[D_OUT:],
      bo1.reshape(1, -1), Wo2, bo2.reshape(1, -1), gamma.reshape(1, -1),
      beta.reshape(1, -1))


def kernel(x, edge_index, edge_attr, pos, Wn1, bn1, Wn2, bn2, We1, be1, We2,
           be2, Wo1, bo1, Wo2, bo2, gamma, beta):
    row = edge_index[0]
    col = edge_index[1]
    xt_lo, xt_hi = _node_mlp(x, Wn1, bn1, Wn2, bn2)
    rel = jnp.take(pos, row, axis=0) - jnp.take(pos, col, axis=0)
    em_lo, em_hi = _edge_msg(rel, edge_attr, We1, be1, We2, be2)
    agg_lo, agg_hi, cnt = _sc_aggregate(em_lo, em_hi, xt_lo, xt_hi, row, col)
    return _out_mlp(agg_lo, agg_hi, cnt, x, Wo1, bo1, Wo2, bo2, gamma, beta)


# speedup vs baseline: 4.3171x; 2.8004x over previous
import functools

import jax
import jax.numpy as jnp
from jax import lax
from jax.experimental import pallas as pl
from jax.experimental.pallas import tpu as pltpu
from jax.experimental.pallas import tpu_sc as plsc

N = 10000
E = 320000
D_IN = 128
D_OUT = 256
DH = 128
D_EDGE = 16

NS = 16
EPT = E // NS
C = 80
NCHUNK = EPT // C
NPT = 640
NPT_STRIDE = 624

_S3 = 3.0 ** 0.5
_S5 = 5.0 ** 0.5
_S15 = 15.0 ** 0.5


def _node_mlp_body(x_ref, w1_ref, b1_ref, w2_ref, b2_ref, lo_ref, hi_ref):
    h = jax.nn.silu(
        jnp.dot(x_ref[...], w1_ref[...], preferred_element_type=jnp.float32)
        + b1_ref[...]
    )
    o = jnp.dot(h, w2_ref[...], preferred_element_type=jnp.float32) + b2_ref[...]
    lo_ref[...] = o[:, :DH]
    hi_ref[...] = o[:, DH:]


def _node_mlp(x, Wn1, bn1, Wn2, bn2):
    return pl.pallas_call(
        _node_mlp_body,
        out_shape=(
            jax.ShapeDtypeStruct((N, DH), jnp.float32),
            jax.ShapeDtypeStruct((N, DH), jnp.float32),
        ),
    )(x, Wn1, bn1.reshape(1, -1), Wn2, bn2.reshape(1, -1))


CC = 400
EPW = E // 32


def _sc_sh_body(posx, posy, posz, rowi, coli,
                shT_o, cnt2_o,
                px_v, py_v, pz_v, ridx_v, cidx_v, sh_v, hist_v,
                hred_v, cdr_v, hist_sh, sem):
    cid = lax.axis_index("c")
    sid = lax.axis_index("s")
    wid = cid * NS + sid
    base = wid * EPW

    pltpu.sync_copy(posx, px_v)
    pltpu.sync_copy(posy, py_v)
    pltpu.sync_copy(posz, pz_v)

    zeros16 = jnp.zeros((16,), jnp.float32)
    ones16 = jnp.ones((16,), jnp.float32)

    def zh(i, _):
        hist_v[pl.ds(i * 16, 16)] = zeros16
        return 0

    lax.fori_loop(0, N // 16, zh, 0)
    for k in range(9, 16):
        for j in range(CC // 16):
            sh_v[k, pl.ds(j * 16, 16)] = zeros16

    def chunk(s, _):
        e0 = base + s * CC
        pltpu.sync_copy(rowi.at[pl.ds(e0, CC)], ridx_v)
        pltpu.sync_copy(coli.at[pl.ds(e0, CC)], cidx_v)

        def inner(j, _):
            sl = pl.ds(j * 16, 16)
            ri = ridx_v[sl]
            ci = cidx_v[sl]
            rx = plsc.load_gather(px_v, [ri]) - plsc.load_gather(px_v, [ci])
            ry = plsc.load_gather(py_v, [ri]) - plsc.load_gather(py_v, [ci])
            rz = plsc.load_gather(pz_v, [ri]) - plsc.load_gather(pz_v, [ci])
            r2 = jnp.maximum(rx * rx + ry * ry + rz * rz, 1e-24)
            ib = plsc.bitcast(r2, jnp.int32)
            ib = 0x5F3759DF - lax.shift_right_logical(ib, 1)
            y = plsc.bitcast(ib, jnp.float32)
            y = y * (1.5 - 0.5 * r2 * y * y)
            y = y * (1.5 - 0.5 * r2 * y * y)
            y = y * (1.5 - 0.5 * r2 * y * y)
            t = r2 * y + 1e-8
            z = y * (2.0 - t * y)
            z = z * (2.0 - t * z)
            dx = rx * z
            dy = ry * z
            dz = rz * z
            sh_v[0, sl] = ones16
            sh_v[1, sl] = _S3 * dx
            sh_v[2, sl] = _S3 * dy
            sh_v[3, sl] = _S3 * dz
            sh_v[4, sl] = _S15 * dx * dy
            sh_v[5, sl] = _S15 * dy * dz
            sh_v[6, sl] = (_S5 / 2.0) * (3.0 * dz * dz - 1.0)
            sh_v[7, sl] = _S15 * dx * dz
            sh_v[8, sl] = (_S15 / 2.0) * (dx * dx - dy * dy)
            plsc.addupdate_scatter(hist_v, [ci], ones16)
            return 0

        lax.fori_loop(0, CC // 16, inner, 0)
        pltpu.sync_copy(sh_v, shT_o.at[:, pl.ds(e0, CC)])
        return 0

    lax.fori_loop(0, EPW // CC, chunk, 0)

    pltpu.sync_copy(hist_v, hist_sh.at[sid])
    plsc.subcore_barrier()
    r0 = sid * NPT_STRIDE
    pltpu.sync_copy(hist_sh.at[:, pl.ds(r0, NPT)], hred_v)

    def rsum(i, _):
        sl = pl.ds(i * 16, 16)
        acc = hred_v[0, sl]
        for r in range(1, NS):
            acc = acc + hred_v[r, sl]
        cdr_v[sl] = acc
        return 0

    lax.fori_loop(0, NPT // 16, rsum, 0)
    pltpu.sync_copy(cdr_v, cnt2_o.at[cid, pl.ds(r0, NPT)])


def _sc_sh(posx, posy, posz, row, col):
    f = pl.kernel(
        _sc_sh_body,
        out_type=(
            jax.ShapeDtypeStruct((16, E), jnp.float32),
            jax.ShapeDtypeStruct((2, N), jnp.float32),
        ),
        mesh=plsc.VectorSubcoreMesh(core_axis_name="c", subcore_axis_name="s"),
        compiler_params=pltpu.CompilerParams(use_tc_tiling_on_sc=False, needs_layout_passes=False),
        scratch_types=[
            pltpu.VMEM((N,), jnp.float32),
            pltpu.VMEM((N,), jnp.float32),
            pltpu.VMEM((N,), jnp.float32),
            pltpu.VMEM((CC,), jnp.int32),
            pltpu.VMEM((CC,), jnp.int32),
            pltpu.VMEM((16, CC), jnp.float32),
            pltpu.VMEM((N,), jnp.float32),
            pltpu.VMEM((NS, NPT), jnp.float32),
            pltpu.VMEM((NPT,), jnp.float32),
            pltpu.VMEM_SHARED((NS, N), jnp.float32),
            pltpu.SemaphoreType.DMA,
        ],
    )
    return f(posx, posy, posz, row, col)


def _edge_body(shT_ref, ea_ref, w1a_ref, w1b_ref, b1_ref, w2_ref, b2_ref,
               lo_ref, hi_ref):
    h = lax.dot_general(shT_ref[...], w1a_ref[...],
                        dimension_numbers=(((0,), (0,)), ((), ())),
                        preferred_element_type=jnp.float32)
    h = h + jnp.dot(ea_ref[...], w1b_ref[...], preferred_element_type=jnp.float32)
    h = jax.nn.silu(h + b1_ref[...])
    em = jnp.dot(h, w2_ref[...], preferred_element_type=jnp.float32) + b2_ref[...]
    lo_ref[...] = em[:, :DH]
    hi_ref[...] = em[:, DH:]


def _edge_msg(shT, edge_attr, We1, be1, We2, be2):
    B = 2560
    grid = E // B
    return pl.pallas_call(
        _edge_body,
        grid=(grid,),
        in_specs=[
            pl.BlockSpec((16, B), lambda i: (0, i)),
            pl.BlockSpec((B, D_EDGE), lambda i: (i, 0)),
            pl.BlockSpec((16, D_OUT), lambda i: (0, 0)),
            pl.BlockSpec((D_EDGE, D_OUT), lambda i: (0, 0)),
            pl.BlockSpec((1, D_OUT), lambda i: (0, 0)),
            pl.BlockSpec((D_OUT, D_OUT), lambda i: (0, 0)),
            pl.BlockSpec((1, D_OUT), lambda i: (0, 0)),
        ],
        out_specs=(
            pl.BlockSpec((B, DH), lambda i: (i, 0)),
            pl.BlockSpec((B, DH), lambda i: (i, 0)),
        ),
        out_shape=(
            jax.ShapeDtypeStruct((E, DH), jnp.float32),
            jax.ShapeDtypeStruct((E, DH), jnp.float32),
        ),
    )(shT, edge_attr,
      jnp.pad(We1[:9], ((0, 7), (0, 0))), We1[9:], be1.reshape(1, -1),
      We2, be2.reshape(1, -1))


def _sc_body(emlo, emhi, xtlo, xthi, rowi, coli,
             agglo, agghi,
             em_v, xt_v, ridx_v, cidx_v,
             agg_sh, gsem):
    cid = lax.axis_index("c")
    sid = lax.axis_index("s")
    r0 = sid * NPT_STRIDE

    zeros16 = jnp.zeros((16,), jnp.float32)

    def zrow(i, _):
        for j in range(DH // 16):
            em_v[i, pl.ds(j * 16, 16)] = zeros16
        return 0

    lax.fori_loop(0, C, zrow, 0)

    for j in range(NPT // C):
        pltpu.sync_copy(em_v, agg_sh.at[pl.ds(r0 + j * C, C)])
    plsc.subcore_barrier()

    def run(em_ref, xt_ref, agg_out):
        base = sid * EPT

        def chunk(k, _):
            e0 = base + k * C
            pltpu.sync_copy(rowi.at[pl.ds(e0, C)], ridx_v)
            pltpu.sync_copy(coli.at[pl.ds(e0, C)], cidx_v)
            g = pltpu.async_copy(xt_ref.at[ridx_v], xt_v, gsem)
            pltpu.sync_copy(em_ref.at[pl.ds(e0, C)], em_v)
            g.wait()

            def mrow(i, _):
                for j in range(DH // 16):
                    s = (i, pl.ds(j * 16, 16))
                    em_v[s] = em_v[s] * xt_v[s]
                return 0

            lax.fori_loop(0, C, mrow, 0)
            pltpu.sync_copy(em_v, agg_sh.at[cidx_v], add=True)
            return 0

        lax.fori_loop(0, NCHUNK, chunk, 0)
        plsc.subcore_barrier()
        for j in range(NPT // C):
            pltpu.sync_copy(agg_sh.at[pl.ds(r0 + j * C, C)], em_v)
            pltpu.sync_copy(em_v, agg_out.at[pl.ds(r0 + j * C, C)])

    @pl.when(cid == 0)
    def _():
        run(emlo, xtlo, agglo)

    @pl.when(cid == 1)
    def _():
        run(emhi, xthi, agghi)


def _sc_aggregate(em_lo, em_hi, xt_lo, xt_hi, row, col):
    f = pl.kernel(
        _sc_body,
        out_type=(
            jax.ShapeDtypeStruct((N, DH), jnp.float32),
            jax.ShapeDtypeStruct((N, DH), jnp.float32),
        ),
        mesh=plsc.VectorSubcoreMesh(core_axis_name="c", subcore_axis_name="s"),
        compiler_params=pltpu.CompilerParams(use_tc_tiling_on_sc=False, needs_layout_passes=False),
        scratch_types=[
            pltpu.VMEM((C, DH), jnp.float32),
            pltpu.VMEM((C, DH), jnp.float32),
            pltpu.VMEM((C,), jnp.int32),
            pltpu.VMEM((C,), jnp.int32),
            pltpu.VMEM_SHARED((N, DH), jnp.float32),
            pltpu.SemaphoreType.DMA,
        ],
    )
    return f(em_lo, em_hi, xt_lo, xt_hi, row, col)


def _out_body(agglo_ref, agghi_ref, cnt_ref, x_ref, w1a_ref, w1b_ref,
              w1c_ref, b1_ref, w2_ref, b2_ref, g_ref, bt_ref, out_ref):
    inv_cnt = 1.0 / jnp.maximum(cnt_ref[...], 1.0)
    alo = agglo_ref[...] * inv_cnt
    ahi = agghi_ref[...] * inv_cnt
    h = jax.nn.silu(
        jnp.dot(alo, w1a_ref[...], preferred_element_type=jnp.float32)
        + jnp.dot(ahi, w1b_ref[...], preferred_element_type=jnp.float32)
        + jnp.dot(x_ref[...], w1c_ref[...], preferred_element_type=jnp.float32)
        + b1_ref[...]
    )
    h = jnp.dot(h, w2_ref[...], preferred_element_type=jnp.float32) + b2_ref[...]
    mu = jnp.mean(h, axis=0, keepdims=True)
    var = jnp.mean((h - mu) ** 2, axis=0, keepdims=True)
    out_ref[...] = (h - mu) * lax.rsqrt(var + 1e-5) * g_ref[...] + bt_ref[...]


def _out_mlp(agg_lo, agg_hi, cnt, x, Wo1, bo1, Wo2, bo2, gamma, beta):
    return pl.pallas_call(
        _out_body,
        out_shape=jax.ShapeDtypeStruct((N, D_OUT), jnp.float32),
    )(agg_lo, agg_hi, cnt, x, Wo1[:DH], Wo1[DH:D_OUT], Wo1[D_OUT:],
      bo1.reshape(1, -1), Wo2, bo2.reshape(1, -1), gamma.reshape(1, -1),
      beta.reshape(1, -1))


def kernel(x, edge_index, edge_attr, pos, Wn1, bn1, Wn2, bn2, We1, be1, We2,
           be2, Wo1, bo1, Wo2, bo2, gamma, beta):
    row = edge_index[0]
    col = edge_index[1]
    xt_lo, xt_hi = _node_mlp(x, Wn1, bn1, Wn2, bn2)
    shT, cnt2 = _sc_sh(pos[:, 0], pos[:, 1], pos[:, 2], row, col)
    em_lo, em_hi = _edge_msg(shT, edge_attr, We1, be1, We2, be2)
    agg_lo, agg_hi = _sc_aggregate(em_lo, em_hi, xt_lo, xt_hi, row, col)
    cnt = (cnt2[0] + cnt2[1]).reshape(N, 1)
    return _out_mlp(agg_lo, agg_hi, cnt, x, Wo1, bo1, Wo2, bo2, gamma, beta)


# speedup vs baseline: 6.4113x; 1.4851x over previous
import functools

import jax
import jax.numpy as jnp
from jax import lax
from jax.experimental import pallas as pl
from jax.experimental.pallas import tpu as pltpu
from jax.experimental.pallas import tpu_sc as plsc

N = 10000
E = 320000
D_IN = 128
D_OUT = 256
DH = 128
D_EDGE = 16

NS = 16
EPT = E // NS
C = 80
NCHUNK = EPT // C
NPT = 640
NPT_STRIDE = 624

_S3 = 3.0 ** 0.5
_S5 = 5.0 ** 0.5
_S15 = 15.0 ** 0.5


def _node_mlp_body(x_ref, w1_ref, b1_ref, w2_ref, b2_ref, lo_ref, hi_ref):
    h = jax.nn.silu(
        jnp.dot(x_ref[...], w1_ref[...], preferred_element_type=jnp.float32)
        + b1_ref[...]
    )
    o = jnp.dot(h, w2_ref[...], preferred_element_type=jnp.float32) + b2_ref[...]
    lo_ref[...] = o[:, :DH]
    hi_ref[...] = o[:, DH:]


def _node_mlp(x, Wn1, bn1, Wn2, bn2):
    return pl.pallas_call(
        _node_mlp_body,
        out_shape=(
            jax.ShapeDtypeStruct((N, DH), jnp.float32),
            jax.ShapeDtypeStruct((N, DH), jnp.float32),
        ),
    )(x, Wn1, bn1.reshape(1, -1), Wn2, bn2.reshape(1, -1))


CC = 400
EPW = E // 32


def _sc_sh_body(posx, posy, posz, rowi, coli,
                shT_o, cnt2_o,
                px_v, py_v, pz_v, ridx_v, cidx_v, sh_v, hist_v,
                hred_v, cdr_v, hist_sh, sem):
    cid = lax.axis_index("c")
    sid = lax.axis_index("s")
    wid = cid * NS + sid
    base = wid * EPW

    pltpu.sync_copy(posx, px_v)
    pltpu.sync_copy(posy, py_v)
    pltpu.sync_copy(posz, pz_v)

    zeros16 = jnp.zeros((16,), jnp.float32)
    ones16 = jnp.ones((16,), jnp.float32)

    def zh(i, _):
        hist_v[pl.ds(i * 16, 16)] = zeros16
        return 0

    lax.fori_loop(0, N // 16, zh, 0)
    for k in range(9, 16):
        for j in range(CC // 16):
            sh_v[k, pl.ds(j * 16, 16)] = zeros16

    def chunk(s, _):
        e0 = base + s * CC
        pltpu.sync_copy(rowi.at[pl.ds(e0, CC)], ridx_v)
        pltpu.sync_copy(coli.at[pl.ds(e0, CC)], cidx_v)

        def inner(j, _):
            sl = pl.ds(j * 16, 16)
            ri = ridx_v[sl]
            ci = cidx_v[sl]
            rx = plsc.load_gather(px_v, [ri]) - plsc.load_gather(px_v, [ci])
            ry = plsc.load_gather(py_v, [ri]) - plsc.load_gather(py_v, [ci])
            rz = plsc.load_gather(pz_v, [ri]) - plsc.load_gather(pz_v, [ci])
            r2 = jnp.maximum(rx * rx + ry * ry + rz * rz, 1e-24)
            ib = plsc.bitcast(r2, jnp.int32)
            ib = 0x5F3759DF - lax.shift_right_logical(ib, 1)
            y = plsc.bitcast(ib, jnp.float32)
            y = y * (1.5 - 0.5 * r2 * y * y)
            y = y * (1.5 - 0.5 * r2 * y * y)
            y = y * (1.5 - 0.5 * r2 * y * y)
            t = r2 * y + 1e-8
            z = y * (2.0 - t * y)
            z = z * (2.0 - t * z)
            dx = rx * z
            dy = ry * z
            dz = rz * z
            sh_v[0, sl] = ones16
            sh_v[1, sl] = _S3 * dx
            sh_v[2, sl] = _S3 * dy
            sh_v[3, sl] = _S3 * dz
            sh_v[4, sl] = _S15 * dx * dy
            sh_v[5, sl] = _S15 * dy * dz
            sh_v[6, sl] = (_S5 / 2.0) * (3.0 * dz * dz - 1.0)
            sh_v[7, sl] = _S15 * dx * dz
            sh_v[8, sl] = (_S15 / 2.0) * (dx * dx - dy * dy)
            plsc.addupdate_scatter(hist_v, [ci], ones16)
            return 0

        lax.fori_loop(0, CC // 16, inner, 0)
        pltpu.sync_copy(sh_v, shT_o.at[:, pl.ds(e0, CC)])
        return 0

    lax.fori_loop(0, EPW // CC, chunk, 0)

    pltpu.sync_copy(hist_v, hist_sh.at[sid])
    plsc.subcore_barrier()
    r0 = sid * NPT_STRIDE
    pltpu.sync_copy(hist_sh.at[:, pl.ds(r0, NPT)], hred_v)

    def rsum(i, _):
        sl = pl.ds(i * 16, 16)
        acc = hred_v[0, sl]
        for r in range(1, NS):
            acc = acc + hred_v[r, sl]
        cdr_v[sl] = acc
        return 0

    lax.fori_loop(0, NPT // 16, rsum, 0)
    pltpu.sync_copy(cdr_v, cnt2_o.at[cid, pl.ds(r0, NPT)])


def _sc_sh(posx, posy, posz, row, col):
    f = pl.kernel(
        _sc_sh_body,
        out_type=(
            jax.ShapeDtypeStruct((16, E), jnp.float32),
            jax.ShapeDtypeStruct((2, N), jnp.float32),
        ),
        mesh=plsc.VectorSubcoreMesh(core_axis_name="c", subcore_axis_name="s"),
        compiler_params=pltpu.CompilerParams(use_tc_tiling_on_sc=False, needs_layout_passes=False),
        scratch_types=[
            pltpu.VMEM((N,), jnp.float32),
            pltpu.VMEM((N,), jnp.float32),
            pltpu.VMEM((N,), jnp.float32),
            pltpu.VMEM((CC,), jnp.int32),
            pltpu.VMEM((CC,), jnp.int32),
            pltpu.VMEM((16, CC), jnp.float32),
            pltpu.VMEM((N,), jnp.float32),
            pltpu.VMEM((NS, NPT), jnp.float32),
            pltpu.VMEM((NPT,), jnp.float32),
            pltpu.VMEM_SHARED((NS, N), jnp.float32),
            pltpu.SemaphoreType.DMA,
        ],
    )
    return f(posx, posy, posz, row, col)


def _edge_body(shT_ref, ea_ref, w1a_ref, w1b_ref, b1_ref, w2_ref, b2_ref,
               lo_ref, hi_ref):
    h = lax.dot_general(shT_ref[...], w1a_ref[...],
                        dimension_numbers=(((0,), (0,)), ((), ())),
                        preferred_element_type=jnp.float32)
    h = h + jnp.dot(ea_ref[...], w1b_ref[...], preferred_element_type=jnp.float32)
    h = jax.nn.silu(h + b1_ref[...])
    em = jnp.dot(h, w2_ref[...], preferred_element_type=jnp.float32) + b2_ref[...]
    lo_ref[...] = em[:, :DH]
    hi_ref[...] = em[:, DH:]


def _edge_msg(shT, edge_attr, We1, be1, We2, be2):
    B = 2560
    grid = E // B
    return pl.pallas_call(
        _edge_body,
        grid=(grid,),
        in_specs=[
            pl.BlockSpec((16, B), lambda i: (0, i)),
            pl.BlockSpec((B, D_EDGE), lambda i: (i, 0)),
            pl.BlockSpec((16, D_OUT), lambda i: (0, 0)),
            pl.BlockSpec((D_EDGE, D_OUT), lambda i: (0, 0)),
            pl.BlockSpec((1, D_OUT), lambda i: (0, 0)),
            pl.BlockSpec((D_OUT, D_OUT), lambda i: (0, 0)),
            pl.BlockSpec((1, D_OUT), lambda i: (0, 0)),
        ],
        out_specs=(
            pl.BlockSpec((B, DH), lambda i: (i, 0)),
            pl.BlockSpec((B, DH), lambda i: (i, 0)),
        ),
        out_shape=(
            jax.ShapeDtypeStruct((E, DH), jnp.float32),
            jax.ShapeDtypeStruct((E, DH), jnp.float32),
        ),
    )(shT, edge_attr,
      jnp.pad(We1[:9], ((0, 7), (0, 0))), We1[9:], be1.reshape(1, -1),
      We2, be2.reshape(1, -1))


SUP = 10


def _sc_body(emlo, emhi, xtlo, xthi, rowi2, coli2,
             agglo, agghi,
             em0, em1, xt0, xt1, ridx_b, cidx_b,
             agg_sh, esem0, esem1, gsem0, gsem1):
    cid = lax.axis_index("c")
    sid = lax.axis_index("s")
    r0 = sid * NPT_STRIDE

    zeros16 = jnp.zeros((16,), jnp.float32)

    def zrow(i, _):
        for j in range(DH // 16):
            em0[i, pl.ds(j * 16, 16)] = zeros16
        return 0

    lax.fori_loop(0, C, zrow, 0)

    for j in range(NPT // C):
        pltpu.sync_copy(em0, agg_sh.at[pl.ds(r0 + j * C, C)])
    plsc.subcore_barrier()

    def run(em_ref, xt_ref, agg_out):
        ems = (em0, em1)
        xts = (xt0, xt1)
        esems = (esem0, esem1)
        gsems = (gsem0, gsem1)
        base_chunk = sid * NCHUNK

        def super_chunk(s, _):
            c0 = base_chunk + s * SUP
            pltpu.sync_copy(rowi2.at[pl.ds(c0, SUP)], ridx_b)
            pltpu.sync_copy(coli2.at[pl.ds(c0, SUP)], cidx_b)

            def issue(j):
                b = j & 1
                e = pltpu.async_copy(
                    em_ref.at[pl.ds((c0 + j) * C, C)], ems[b], esems[b])
                g = pltpu.async_copy(
                    xt_ref.at[ridx_b.at[j]], xts[b], gsems[b])
                return e, g

            pend = issue(0)
            for j in range(SUP):
                b = j & 1
                nxt = issue(j + 1) if j + 1 < SUP else None
                pend[0].wait()
                pend[1].wait()

                def mrow(i, _):
                    for q in range(DH // 16):
                        sq = (i, pl.ds(q * 16, 16))
                        ems[b][sq] = ems[b][sq] * xts[b][sq]
                    return 0

                lax.fori_loop(0, C, mrow, 0)
                pltpu.sync_copy(ems[b], agg_sh.at[cidx_b.at[j]], add=True)
                pend = nxt
            return 0

        lax.fori_loop(0, NCHUNK // SUP, super_chunk, 0)
        plsc.subcore_barrier()
        for j in range(NPT // C):
            pltpu.sync_copy(agg_sh.at[pl.ds(r0 + j * C, C)], em0)
            pltpu.sync_copy(em0, agg_out.at[pl.ds(r0 + j * C, C)])

    @pl.when(cid == 0)
    def _():
        run(emlo, xtlo, agglo)

    @pl.when(cid == 1)
    def _():
        run(emhi, xthi, agghi)


def _sc_aggregate(em_lo, em_hi, xt_lo, xt_hi, row2, col2):
    f = pl.kernel(
        _sc_body,
        out_type=(
            jax.ShapeDtypeStruct((N, DH), jnp.float32),
            jax.ShapeDtypeStruct((N, DH), jnp.float32),
        ),
        mesh=plsc.VectorSubcoreMesh(core_axis_name="c", subcore_axis_name="s"),
        compiler_params=pltpu.CompilerParams(use_tc_tiling_on_sc=False, needs_layout_passes=False),
        scratch_types=[
            pltpu.VMEM((C, DH), jnp.float32),
            pltpu.VMEM((C, DH), jnp.float32),
            pltpu.VMEM((C, DH), jnp.float32),
            pltpu.VMEM((C, DH), jnp.float32),
            pltpu.VMEM((SUP, C), jnp.int32),
            pltpu.VMEM((SUP, C), jnp.int32),
            pltpu.VMEM_SHARED((N, DH), jnp.float32),
            pltpu.SemaphoreType.DMA,
            pltpu.SemaphoreType.DMA,
            pltpu.SemaphoreType.DMA,
            pltpu.SemaphoreType.DMA,
        ],
    )
    return f(em_lo, em_hi, xt_lo, xt_hi, row2, col2)


def _out_body(agglo_ref, agghi_ref, cnt_ref, x_ref, w1a_ref, w1b_ref,
              w1c_ref, b1_ref, w2_ref, b2_ref, g_ref, bt_ref, out_ref):
    inv_cnt = 1.0 / jnp.maximum(cnt_ref[...], 1.0)
    alo = agglo_ref[...] * inv_cnt
    ahi = agghi_ref[...] * inv_cnt
    h = jax.nn.silu(
        jnp.dot(alo, w1a_ref[...], preferred_element_type=jnp.float32)
        + jnp.dot(ahi, w1b_ref[...], preferred_element_type=jnp.float32)
        + jnp.dot(x_ref[...], w1c_ref[...], preferred_element_type=jnp.float32)
        + b1_ref[...]
    )
    h = jnp.dot(h, w2_ref[...], preferred_element_type=jnp.float32) + b2_ref[...]
    mu = jnp.mean(h, axis=0, keepdims=True)
    var = jnp.mean((h - mu) ** 2, axis=0, keepdims=True)
    out_ref[...] = (h - mu) * lax.rsqrt(var + 1e-5) * g_ref[...] + bt_ref[...]


def _out_mlp(agg_lo, agg_hi, cnt, x, Wo1, bo1, Wo2, bo2, gamma, beta):
    return pl.pallas_call(
        _out_body,
        out_shape=jax.ShapeDtypeStruct((N, D_OUT), jnp.float32),
    )(agg_lo, agg_hi, cnt, x, Wo1[:DH], Wo1[DH:D_OUT], Wo1[D_OUT:],
      bo1.reshape(1, -1), Wo2, bo2.reshape(1, -1), gamma.reshape(1, -1),
      beta.reshape(1, -1))


def kernel(x, edge_index, edge_attr, pos, Wn1, bn1, Wn2, bn2, We1, be1, We2,
           be2, Wo1, bo1, Wo2, bo2, gamma, beta):
    row = edge_index[0]
    col = edge_index[1]
    xt_lo, xt_hi = _node_mlp(x, Wn1, bn1, Wn2, bn2)
    shT, cnt2 = _sc_sh(pos[:, 0], pos[:, 1], pos[:, 2], row, col)
    em_lo, em_hi = _edge_msg(shT, edge_attr, We1, be1, We2, be2)
    agg_lo, agg_hi = _sc_aggregate(em_lo, em_hi, xt_lo, xt_hi,
                                   row.reshape(E // C, C),
                                   col.reshape(E // C, C))
    cnt = (cnt2[0] + cnt2[1]).reshape(N, 1)
    return _out_mlp(agg_lo, agg_hi, cnt, x, Wo1, bo1, Wo2, bo2, gamma, beta)
